# baseline (speedup 1.0000x reference)
import functools
import math

import jax
import jax.numpy as jnp
from jax import lax
from jax.experimental import pallas as pl
from jax.experimental.pallas import tpu as pltpu
from jax.experimental.pallas import tpu_sc as plsc

NC = 2
NS = 16
NW = NC * NS
L = 16
NBUF = 3


@functools.partial(jax.jit, static_argnames=("B", "S", "D", "Cs"))
def _posemb_sc(x, word_table, pos_table, B, S, D, Cs):
    N = B * S
    s_per_w = S // NW
    n_chunks = s_per_w // Cs
    R = B * Cs
    scale = float(math.sqrt(D))
    vregs = D // L

    mesh = plsc.VectorSubcoreMesh(core_axis_name="c", subcore_axis_name="s")

    @functools.partial(
        pl.kernel,
        mesh=mesh,
        out_type=jax.ShapeDtypeStruct((N, D), jnp.float32),
        scratch_types=[
            pltpu.VMEM((B, s_per_w), jnp.int32),
            pltpu.VMEM((NBUF, R, D), jnp.float32),
            pltpu.VMEM((NBUF, Cs, D), jnp.float32),
        ] + [pltpu.SemaphoreType.DMA] * (3 * NBUF),
    )
    def run(x_hbm, word_hbm, pos_hbm, out_hbm, xbuf, wbuf, pbuf, *sems):
        wsems = sems[0:NBUF]
        psems = sems[NBUF:2 * NBUF]
        osems = sems[2 * NBUF:3 * NBUF]
        cid = lax.axis_index("c")
        sid = lax.axis_index("s")
        wid = sid * NC + cid
        s_base = wid * s_per_w

        for b in range(B):
            pltpu.sync_copy(x_hbm.at[b, pl.ds(s_base, s_per_w)], xbuf.at[b])

        def start(g):
            slot = g % NBUF
            hs = [pltpu.async_copy(
                      word_hbm.at[xbuf.at[b, pl.ds(g * Cs, Cs)]],
                      wbuf.at[slot, pl.ds(b * Cs, Cs)],
                      wsems[slot]) for b in range(B)]
            hs.append(pltpu.async_copy(
                pos_hbm.at[pl.ds(s_base + g * Cs, Cs)],
                pbuf.at[slot], psems[slot]))
            return hs

        owaits = [None] * NBUF
        starts = {0: start(0)}
        if n_chunks > 1:
            starts[1] = start(1)
        for g in range(n_chunks):
            slot = g % NBUF
            if g + 2 < n_chunks:
                ns = (g + 2) % NBUF
                for h in owaits[ns] or ():
                    h.wait()
                owaits[ns] = None
                starts[g + 2] = start(g + 2)
            for h in starts.pop(g):
                h.wait()

            @plsc.parallel_loop(0, R * vregs, unroll=8)
            def _(k):
                r = k // vregs
                j = (k % vregs) * L
                i = r % Cs
                w = wbuf[slot, r, pl.ds(j, L)]
                p = pbuf[slot, i, pl.ds(j, L)]
                wbuf[slot, r, pl.ds(j, L)] = w * scale + p

            owaits[slot] = [pltpu.async_copy(
                wbuf.at[slot, pl.ds(b * Cs, Cs)],
                out_hbm.at[pl.ds(b * S + s_base + g * Cs, Cs)],
                osems[slot]) for b in range(B)]
        for hs in owaits:
            for h in hs or ():
                h.wait()

    return run(x, word_table, pos_table)


def kernel(x, word_table, pos_table):
    B, S = x.shape
    V, D = word_table.shape
    Cs = 8
    assert S % (NW * Cs) == 0 and D % L == 0
    out = _posemb_sc(x.astype(jnp.int32), word_table, pos_table, B, S, D, Cs)
    return out.reshape(B, S, D)


# speedup vs baseline: 1.0283x; 1.0283x over previous
import functools
import math

import jax
import jax.numpy as jnp
from jax import lax
from jax.experimental import pallas as pl
from jax.experimental.pallas import tpu as pltpu
from jax.experimental.pallas import tpu_sc as plsc

NC = 2
NS = 16
NW = NC * NS
L = 16
NBUF = 3


@functools.partial(jax.jit, static_argnames=("B", "S", "D", "Cs"))
def _posemb_sc(x, word_table, pos_table, B, S, D, Cs):
    N = B * S
    s_per_w = S // NW
    n_chunks = s_per_w // Cs
    R = B * Cs
    scale = float(math.sqrt(D))
    vregs = D // L

    mesh = plsc.VectorSubcoreMesh(core_axis_name="c", subcore_axis_name="s")

    @functools.partial(
        pl.kernel,
        mesh=mesh,
        out_type=jax.ShapeDtypeStruct((N, D), jnp.float32),
        scratch_types=[
            pltpu.VMEM((n_chunks, R), jnp.int32),
            pltpu.VMEM((NBUF, R, D), jnp.float32),
            pltpu.VMEM((NBUF, Cs, D), jnp.float32),
        ] + [pltpu.SemaphoreType.DMA] * (3 * NBUF),
    )
    def run(idx_hbm, word_hbm, pos_hbm, out_hbm, idx_v, wbuf, pbuf, *sems):
        wsems = sems[0:NBUF]
        psems = sems[NBUF:2 * NBUF]
        osems = sems[2 * NBUF:3 * NBUF]
        cid = lax.axis_index("c")
        sid = lax.axis_index("s")
        wid = sid * NC + cid
        s_base = wid * s_per_w

        pltpu.sync_copy(idx_hbm.at[wid], idx_v)

        def start(g):
            slot = g % NBUF
            return [
                pltpu.async_copy(word_hbm.at[idx_v.at[g]], wbuf.at[slot],
                                 wsems[slot]),
                pltpu.async_copy(pos_hbm.at[pl.ds(s_base + g * Cs, Cs)],
                                 pbuf.at[slot], psems[slot]),
            ]

        owaits = [None] * NBUF
        starts = {0: start(0)}
        if n_chunks > 1:
            starts[1] = start(1)
        for g in range(n_chunks):
            slot = g % NBUF
            if g + 2 < n_chunks:
                ns = (g + 2) % NBUF
                for h in owaits[ns] or ():
                    h.wait()
                owaits[ns] = None
                starts[g + 2] = start(g + 2)
            for h in starts.pop(g):
                h.wait()

            @plsc.parallel_loop(0, R * vregs, unroll=8)
            def _(k):
                r = k // vregs
                j = (k % vregs) * L
                i = r % Cs
                w = wbuf[slot, r, pl.ds(j, L)]
                p = pbuf[slot, i, pl.ds(j, L)]
                wbuf[slot, r, pl.ds(j, L)] = w * scale + p

            owaits[slot] = [pltpu.async_copy(
                wbuf.at[slot, pl.ds(b * Cs, Cs)],
                out_hbm.at[pl.ds(b * S + s_base + g * Cs, Cs)],
                osems[slot]) for b in range(B)]
        for hs in owaits:
            for h in hs or ():
                h.wait()

    return run(x, word_table, pos_table)


def kernel(x, word_table, pos_table):
    B, S = x.shape
    V, D = word_table.shape
    Cs = 8
    s_per_w = S // NW
    assert S % (NW * Cs) == 0 and D % L == 0
    idx4 = (x.reshape(B, NW, s_per_w // Cs, Cs)
              .transpose(1, 2, 0, 3)
              .reshape(NW, s_per_w // Cs, B * Cs)
              .astype(jnp.int32))
    out = _posemb_sc(idx4, word_table, pos_table, B, S, D, Cs)
    return out.reshape(B, S, D)


# speedup vs baseline: 1.0812x; 1.0515x over previous
import functools
import math

import jax
import jax.numpy as jnp
from jax import lax
from jax.experimental import pallas as pl
from jax.experimental.pallas import tpu as pltpu
from jax.experimental.pallas import tpu_sc as plsc

NC = 2
NS = 16
NW = NC * NS
L = 16
NBUF = 3


@functools.partial(jax.jit, static_argnames=("B", "S", "D", "Cs"))
def _posemb_sc(x, word_table, pos_table, B, S, D, Cs):
    N = B * S
    s_per_w = S // NW
    n_chunks = s_per_w // Cs
    R = B * Cs
    scale = float(math.sqrt(D))
    vregs = D // L

    mesh = plsc.VectorSubcoreMesh(core_axis_name="c", subcore_axis_name="s")

    @functools.partial(
        pl.kernel,
        mesh=mesh,
        out_type=jax.ShapeDtypeStruct((N, D), jnp.float32),
        scratch_types=[
            pltpu.VMEM((n_chunks, R), jnp.int32),
            pltpu.VMEM((NBUF, R, D), jnp.float32),
            pltpu.VMEM((NBUF, Cs, D), jnp.float32),
        ] + [pltpu.SemaphoreType.DMA] * (3 * NBUF),
    )
    def run(idx_hbm, word_hbm, pos_hbm, out_hbm, idx_v, wbuf, pbuf, *sems):
        wsems = sems[0:NBUF]
        psems = sems[NBUF:2 * NBUF]
        osems = sems[2 * NBUF:3 * NBUF]
        cid = lax.axis_index("c")
        sid = lax.axis_index("s")
        wid = sid * NC + cid
        s_base = wid * s_per_w

        pltpu.sync_copy(idx_hbm.at[wid], idx_v)

        def start(g):
            slot = g % NBUF
            return [
                pltpu.async_copy(word_hbm.at[idx_v.at[g]], wbuf.at[slot],
                                 wsems[slot]),
                pltpu.async_copy(pos_hbm.at[pl.ds(s_base + g * Cs, Cs)],
                                 pbuf.at[slot], psems[slot]),
            ]

        owaits = [None] * NBUF
        starts = {0: start(0)}
        if n_chunks > 1:
            starts[1] = start(1)
        for g in range(n_chunks):
            slot = g % NBUF
            if g + 2 < n_chunks:
                ns = (g + 2) % NBUF
                for h in owaits[ns] or ():
                    h.wait()
                owaits[ns] = None
                starts[g + 2] = start(g + 2)
            for h in starts.pop(g):
                h.wait()

            @plsc.parallel_loop(0, Cs * vregs, unroll=4)
            def _(k):
                i = k // vregs
                j = (k % vregs) * L
                p = pbuf[slot, i, pl.ds(j, L)]
                for b in range(B):
                    w = wbuf[slot, b * Cs + i, pl.ds(j, L)]
                    wbuf[slot, b * Cs + i, pl.ds(j, L)] = w * scale + p

            owaits[slot] = [pltpu.async_copy(
                wbuf.at[slot, pl.ds(b * Cs, Cs)],
                out_hbm.at[pl.ds(b * S + s_base + g * Cs, Cs)],
                osems[slot]) for b in range(B)]
        for hs in owaits:
            for h in hs or ():
                h.wait()

    return run(x, word_table, pos_table)


def kernel(x, word_table, pos_table):
    B, S = x.shape
    V, D = word_table.shape
    Cs = 8
    s_per_w = S // NW
    assert S % (NW * Cs) == 0 and D % L == 0
    idx4 = (x.reshape(B, NW, s_per_w // Cs, Cs)
              .transpose(1, 2, 0, 3)
              .reshape(NW, s_per_w // Cs, B * Cs)
              .astype(jnp.int32))
    out = _posemb_sc(idx4, word_table, pos_table, B, S, D, Cs)
    return out.reshape(B, S, D)


# speedup vs baseline: 1.0910x; 1.0091x over previous
import functools
import math

import jax
import jax.numpy as jnp
from jax import lax
from jax.experimental import pallas as pl
from jax.experimental.pallas import tpu as pltpu
from jax.experimental.pallas import tpu_sc as plsc

NC = 2
NS = 16
NW = NC * NS
L = 16
NBUF = 3


@functools.partial(jax.jit, static_argnames=("B", "S", "D", "Cs"))
def _posemb_sc(x, word_table, pos_table, B, S, D, Cs):
    N = B * S
    s_per_w = S // NW
    n_chunks = s_per_w // Cs
    R = B * Cs
    scale = float(math.sqrt(D))
    vregs = D // L

    mesh = plsc.VectorSubcoreMesh(core_axis_name="c", subcore_axis_name="s")

    @functools.partial(
        pl.kernel,
        mesh=mesh,
        out_type=jax.ShapeDtypeStruct((N, D), jnp.float32),
        scratch_types=[
            pltpu.VMEM((n_chunks, R), jnp.int32),
            pltpu.VMEM((NBUF, R, D), jnp.float32),
            pltpu.VMEM((NBUF, Cs, D), jnp.float32),
        ] + [pltpu.SemaphoreType.DMA] * (3 * NBUF),
    )
    def run(idx_hbm, word_hbm, pos_hbm, out_hbm, idx_v, wbuf, pbuf, *sems):
        wsems = sems[0:NBUF]
        psems = sems[NBUF:2 * NBUF]
        osems = sems[2 * NBUF:3 * NBUF]
        cid = lax.axis_index("c")
        sid = lax.axis_index("s")
        wid = sid * NC + cid
        s_base = wid * s_per_w

        pltpu.sync_copy(idx_hbm.at[wid], idx_v)

        def start(g):
            slot = g % NBUF
            return [
                pltpu.async_copy(word_hbm.at[idx_v.at[g]], wbuf.at[slot],
                                 wsems[slot]),
                pltpu.async_copy(pos_hbm.at[pl.ds(s_base + g * Cs, Cs)],
                                 pbuf.at[slot], psems[slot]),
            ]

        owaits = [None] * NBUF
        starts = {0: start(0)}
        if n_chunks > 1:
            starts[1] = start(1)
        for g in range(n_chunks):
            slot = g % NBUF
            if g + 2 < n_chunks:
                ns = (g + 2) % NBUF
                for h in owaits[ns] or ():
                    h.wait()
                owaits[ns] = None
                starts[g + 2] = start(g + 2)
            for h in starts.pop(g):
                h.wait()

            @plsc.parallel_loop(0, Cs * vregs, unroll=8)
            def _(k):
                i = k // vregs
                j = (k % vregs) * L
                p = pbuf[slot, i, pl.ds(j, L)]
                for b in range(B):
                    w = wbuf[slot, b * Cs + i, pl.ds(j, L)]
                    wbuf[slot, b * Cs + i, pl.ds(j, L)] = w * scale + p

            owaits[slot] = [pltpu.async_copy(
                wbuf.at[slot, pl.ds(b * Cs, Cs)],
                out_hbm.at[pl.ds(b * S + s_base + g * Cs, Cs)],
                osems[slot]) for b in range(B)]
        for hs in owaits:
            for h in hs or ():
                h.wait()

    return run(x, word_table, pos_table)


def kernel(x, word_table, pos_table):
    B, S = x.shape
    V, D = word_table.shape
    Cs = 8
    s_per_w = S // NW
    assert S % (NW * Cs) == 0 and D % L == 0
    idx4 = (x.reshape(B, NW, s_per_w // Cs, Cs)
              .transpose(1, 2, 0, 3)
              .reshape(NW, s_per_w // Cs, B * Cs)
              .astype(jnp.int32))
    out = _posemb_sc(idx4, word_table, pos_table, B, S, D, Cs)
    return out.reshape(B, S, D)
